# Initial kernel scaffold; baseline (speedup 1.0000x reference)
import jax
import jax.numpy as jnp
from jax.experimental import pallas as pl


def kernel(x1, x2, edge_index, W1, b1, W2, b2, Wc1, bc1, Wmu, bmu, Wls, bls):
    raise NotImplementedError("write your pallas kernel here")


# speedup vs baseline: 6.9047x; 6.9047x over previous
import jax
import jax.numpy as jnp
from jax import lax
from jax.experimental import pallas as pl
from jax.experimental.pallas import tpu as pltpu
from jax.experimental.pallas import tpu_sc as plsc

_N = 10000
_E = 320000
_F = 128
_NC = 2
_NS = 16
_NW = _NC * _NS
_P = 10240
_EP = 327680
_EPT = _EP // _NW
_G = 128
_NB = _EPT // _G
_RPT = _P // _NS
_R = 1024


def _sc_prop_body(src_hbm, dst_hbm, q_hbm, zeros_hbm, out_hbm,
                  idx_s, idx_d, rows, accum, sem):
    c = lax.axis_index("c")
    s = lax.axis_index("s")
    wid = s * _NC + c
    r0 = s * _RPT
    pltpu.sync_copy(zeros_hbm.at[pl.ds(r0, _RPT)], accum.at[pl.ds(r0, _RPT)])
    plsc.subcore_barrier()
    ebase = wid * _EPT

    def body(j, carry):
        base = pl.multiple_of(ebase + j * _G, _G)
        pltpu.sync_copy(src_hbm.at[pl.ds(base, _G)], idx_s)
        pltpu.sync_copy(dst_hbm.at[pl.ds(base, _G)], idx_d)
        pltpu.async_copy(q_hbm.at[idx_s], rows, sem).wait()
        pltpu.sync_copy(rows, accum.at[idx_d], add=True)
        return carry

    lax.fori_loop(0, _NB, body, 0)
    plsc.subcore_barrier()
    pltpu.sync_copy(accum.at[pl.ds(r0, _RPT)],
                    out_hbm.at[pl.ds(c * _P + r0, _RPT)])


_sc_prop = pl.kernel(
    _sc_prop_body,
    out_type=jax.ShapeDtypeStruct((_NC * _P, _F), jnp.float32),
    mesh=plsc.VectorSubcoreMesh(core_axis_name="c", subcore_axis_name="s"),
    scratch_types=[
        pltpu.VMEM((_G,), jnp.int32),
        pltpu.VMEM((_G,), jnp.int32),
        pltpu.VMEM((_G, _F), jnp.float32),
        pltpu.VMEM_SHARED((_P, _F), jnp.float32),
        pltpu.SemaphoreType.DMA,
    ],
)


def _sc_deg_body(dst_hbm, zeros_hbm, out_hbm, idx_d, ones_v, accum):
    c = lax.axis_index("c")
    s = lax.axis_index("s")
    wid = s * _NC + c
    for i in range(_G // 16):
        ones_v[pl.ds(i * 16, 16)] = jnp.ones((16,), jnp.float32)
    r0 = s * _RPT
    pltpu.sync_copy(zeros_hbm.at[pl.ds(r0, _RPT)], accum.at[pl.ds(r0, _RPT)])
    plsc.subcore_barrier()
    ebase = wid * _EPT

    def body(j, carry):
        base = pl.multiple_of(ebase + j * _G, _G)
        pltpu.sync_copy(dst_hbm.at[pl.ds(base, _G)], idx_d)
        pltpu.sync_copy(ones_v, accum.at[idx_d], add=True)
        return carry

    lax.fori_loop(0, _NB, body, 0)
    plsc.subcore_barrier()
    pltpu.sync_copy(accum.at[pl.ds(r0, _RPT)],
                    out_hbm.at[pl.ds(c * _P + r0, _RPT)])


_sc_deg = pl.kernel(
    _sc_deg_body,
    out_type=jax.ShapeDtypeStruct((_NC * _P,), jnp.float32),
    mesh=plsc.VectorSubcoreMesh(core_axis_name="c", subcore_axis_name="s"),
    scratch_types=[
        pltpu.VMEM((_G,), jnp.int32),
        pltpu.VMEM((_G,), jnp.float32),
        pltpu.VMEM_SHARED((_P,), jnp.float32),
    ],
)


def _mm1_body(x1_ref, x2_ref, w1_ref, b1_ref, w2_ref, b2_ref, wa_ref, wb_ref,
              p1_ref):
    h1 = jnp.dot(x1_ref[...], w1_ref[...],
                 preferred_element_type=jnp.float32) + b1_ref[...]
    h2 = jnp.dot(x2_ref[...], w2_ref[...],
                 preferred_element_type=jnp.float32) + b2_ref[...]
    p1_ref[...] = (jnp.dot(h1, wa_ref[...], preferred_element_type=jnp.float32)
                   + jnp.dot(h2, wb_ref[...], preferred_element_type=jnp.float32))


_mm1 = pl.pallas_call(
    _mm1_body,
    grid=(_P // _R,),
    in_specs=[
        pl.BlockSpec((_R, _F), lambda i: (i, 0)),
        pl.BlockSpec((_R, _F), lambda i: (i, 0)),
        pl.BlockSpec((_F, _F), lambda i: (0, 0)),
        pl.BlockSpec((1, _F), lambda i: (0, 0)),
        pl.BlockSpec((_F, _F), lambda i: (0, 0)),
        pl.BlockSpec((1, _F), lambda i: (0, 0)),
        pl.BlockSpec((_F, _F), lambda i: (0, 0)),
        pl.BlockSpec((_F, _F), lambda i: (0, 0)),
    ],
    out_specs=pl.BlockSpec((_R, _F), lambda i: (i, 0)),
    out_shape=jax.ShapeDtypeStruct((_P, _F), jnp.float32),
)


def _scale_body(deg0_ref, deg1_ref, p1_ref, q1_ref, inv_ref, inv2_ref):
    deg = deg0_ref[...] + deg1_ref[...] + 1.0
    inv = lax.rsqrt(deg)
    inv_ref[...] = inv
    inv2_ref[...] = inv * inv
    q1_ref[...] = p1_ref[...] * inv


_scale = pl.pallas_call(
    _scale_body,
    grid=(_P // _R,),
    in_specs=[
        pl.BlockSpec((_R, 1), lambda i: (i, 0)),
        pl.BlockSpec((_R, 1), lambda i: (i, 0)),
        pl.BlockSpec((_R, _F), lambda i: (i, 0)),
    ],
    out_specs=[
        pl.BlockSpec((_R, _F), lambda i: (i, 0)),
        pl.BlockSpec((_R, 1), lambda i: (i, 0)),
        pl.BlockSpec((_R, 1), lambda i: (i, 0)),
    ],
    out_shape=[
        jax.ShapeDtypeStruct((_P, _F), jnp.float32),
        jax.ShapeDtypeStruct((_P, 1), jnp.float32),
        jax.ShapeDtypeStruct((_P, 1), jnp.float32),
    ],
)


def _hq_body(r0_ref, r1_ref, p1_ref, inv_ref, inv2_ref, bc1_ref, h_ref,
             q2_ref):
    r = (r0_ref[...] + r1_ref[...]) * inv_ref[...]
    h = jnp.maximum(r + p1_ref[...] * inv2_ref[...] + bc1_ref[...], 0.0)
    h_ref[...] = h
    q2_ref[...] = h * inv_ref[...]


_hq = pl.pallas_call(
    _hq_body,
    grid=(_P // _R,),
    in_specs=[
        pl.BlockSpec((_R, _F), lambda i: (i, 0)),
        pl.BlockSpec((_R, _F), lambda i: (i, 0)),
        pl.BlockSpec((_R, _F), lambda i: (i, 0)),
        pl.BlockSpec((_R, 1), lambda i: (i, 0)),
        pl.BlockSpec((_R, 1), lambda i: (i, 0)),
        pl.BlockSpec((1, _F), lambda i: (0, 0)),
    ],
    out_specs=[
        pl.BlockSpec((_R, _F), lambda i: (i, 0)),
        pl.BlockSpec((_R, _F), lambda i: (i, 0)),
    ],
    out_shape=[
        jax.ShapeDtypeStruct((_P, _F), jnp.float32),
        jax.ShapeDtypeStruct((_P, _F), jnp.float32),
    ],
)


def _out_body(r0_ref, r1_ref, h_ref, inv_ref, inv2_ref, wmu_ref, bmu_ref,
              wls_ref, bls_ref, mu_ref, ls_ref):
    g = (r0_ref[...] + r1_ref[...]) * inv_ref[...] + h_ref[...] * inv2_ref[...]
    mu_ref[...] = jnp.dot(g, wmu_ref[...],
                          preferred_element_type=jnp.float32) + bmu_ref[...]
    ls_ref[...] = jnp.dot(g, wls_ref[...],
                          preferred_element_type=jnp.float32) + bls_ref[...]


_outk = pl.pallas_call(
    _out_body,
    grid=(_P // _R,),
    in_specs=[
        pl.BlockSpec((_R, _F), lambda i: (i, 0)),
        pl.BlockSpec((_R, _F), lambda i: (i, 0)),
        pl.BlockSpec((_R, _F), lambda i: (i, 0)),
        pl.BlockSpec((_R, 1), lambda i: (i, 0)),
        pl.BlockSpec((_R, 1), lambda i: (i, 0)),
        pl.BlockSpec((_F, 64), lambda i: (0, 0)),
        pl.BlockSpec((1, 64), lambda i: (0, 0)),
        pl.BlockSpec((_F, 64), lambda i: (0, 0)),
        pl.BlockSpec((1, 64), lambda i: (0, 0)),
    ],
    out_specs=[
        pl.BlockSpec((_R, 64), lambda i: (i, 0)),
        pl.BlockSpec((_R, 64), lambda i: (i, 0)),
    ],
    out_shape=[
        jax.ShapeDtypeStruct((_P, 64), jnp.float32),
        jax.ShapeDtypeStruct((_P, 64), jnp.float32),
    ],
)


def kernel(x1, x2, edge_index, W1, b1, W2, b2, Wc1, bc1, Wmu, bmu, Wls, bls):
    f32 = jnp.float32
    x1p = jnp.pad(x1.astype(f32), ((0, _P - _N), (0, 0)))
    x2p = jnp.pad(x2.astype(f32), ((0, _P - _N), (0, 0)))
    src = jnp.pad(edge_index[0], (0, _EP - _E), constant_values=_P - 1)
    dst = jnp.pad(edge_index[1], (0, _EP - _E), constant_values=_P - 1)
    zeros_pf = jnp.zeros((_P, _F), f32)
    zeros_p = jnp.zeros((_P,), f32)

    degp = _sc_deg(dst, zeros_p)
    deg0 = degp[:_P].reshape(_P, 1)
    deg1 = degp[_P:].reshape(_P, 1)

    p1 = _mm1(x1p, x2p, W1, b1.reshape(1, -1), W2, b2.reshape(1, -1),
              Wc1[:_F], Wc1[_F:])
    q1, inv, inv2 = _scale(deg0, deg1, p1)
    r1 = _sc_prop(src, dst, q1, zeros_pf)
    h, q2 = _hq(r1[:_P], r1[_P:], p1, inv, inv2, bc1.reshape(1, -1))
    r2 = _sc_prop(src, dst, q2, zeros_pf)
    mu, ls = _outk(r2[:_P], r2[_P:], h, inv, inv2, Wmu, bmu.reshape(1, -1),
                   Wls, bls.reshape(1, -1))
    return mu[:_N], ls[:_N]
